# baseline (speedup 1.0000x reference)
import functools

import jax
import jax.numpy as jnp
from jax import lax
from jax.experimental import pallas as pl
from jax.experimental.pallas import tpu as pltpu
from jax.experimental.pallas import tpu_sc as plsc

BATCH = 16384
D = 64
NC = 2
NS = 16
NW = NC * NS
BPW = BATCH // NW
CH = 128
NCH = BPW // CH
GROUPS = BPW // 16


def _cf_body(uid_hbm, tid_hbm, uemb_hbm, temb_hbm, ubias_hbm, tbias_hbm,
             gbias_hbm, out_hbm,
             uid_v, tid_v, urows, trows, ub_v, tb_v, gb_v, out_v, sem):
  wid = lax.axis_index("s") * NC + lax.axis_index("c")
  base = wid * BPW

  pltpu.sync_copy(uid_hbm.at[pl.ds(base, BPW)], uid_v)
  pltpu.sync_copy(tid_hbm.at[pl.ds(base, BPW)], tid_v)
  pltpu.sync_copy(gbias_hbm, gb_v)

  copies = []
  for j in range(NCH):
    sl = pl.ds(j * CH, CH)
    copies.append(pltpu.async_copy(uemb_hbm.at[uid_v.at[sl]], urows.at[sl], sem))
    copies.append(pltpu.async_copy(temb_hbm.at[tid_v.at[sl]], trows.at[sl], sem))
    copies.append(pltpu.async_copy(ubias_hbm.at[uid_v.at[sl]], ub_v.at[sl], sem))
    copies.append(pltpu.async_copy(tbias_hbm.at[tid_v.at[sl]], tb_v.at[sl], sem))
  for c in copies:
    c.wait()

  lanes = lax.iota(jnp.int32, 16)
  gb = gb_v[...]

  def group_body(g, _):
    row_idx = g * 16 + lanes
    acc = jnp.zeros((16,), jnp.float32)
    for j in range(D):
      col = jnp.bitwise_and(lanes + j, D - 1)
      u = plsc.load_gather(urows, [row_idx, col])
      t = plsc.load_gather(trows, [row_idx, col])
      acc = acc + u * t
    gsl = pl.ds(g * 16, 16)
    out_v[gsl] = acc + ub_v[gsl] + tb_v[gsl] + gb
    return ()

  lax.fori_loop(0, GROUPS, group_body, ())

  pltpu.sync_copy(out_v, out_hbm.at[pl.ds(base, BPW)])


@jax.jit
def _cf_call(uid, tid, uemb, temb, ubias, tbias, gbias):
  mesh = plsc.VectorSubcoreMesh(core_axis_name="c", subcore_axis_name="s")
  kern = functools.partial(
      pl.kernel,
      out_type=jax.ShapeDtypeStruct((BATCH,), jnp.float32),
      mesh=mesh,
      compiler_params=pltpu.CompilerParams(
          needs_layout_passes=False, use_tc_tiling_on_sc=False),
      scratch_types=[
          pltpu.VMEM((BPW,), jnp.int32),
          pltpu.VMEM((BPW,), jnp.int32),
          pltpu.VMEM((BPW, D), jnp.float32),
          pltpu.VMEM((BPW, D), jnp.float32),
          pltpu.VMEM((BPW,), jnp.float32),
          pltpu.VMEM((BPW,), jnp.float32),
          pltpu.VMEM((16,), jnp.float32),
          pltpu.VMEM((BPW,), jnp.float32),
          pltpu.SemaphoreType.DMA,
      ],
  )(_cf_body)
  return kern(uid, tid, uemb, temb, ubias, tbias, gbias)


def kernel(user_ids, track_ids, user_embeddings, track_embeddings,
           user_bias, track_bias, global_bias):
  uid = user_ids.astype(jnp.int32)
  tid = track_ids.astype(jnp.int32)
  ubias = user_bias.reshape(-1)
  tbias = track_bias.reshape(-1)
  gbias = jnp.broadcast_to(global_bias, (16,))
  return _cf_call(uid, tid, user_embeddings, track_embeddings,
                  ubias, tbias, gbias)
